# Initial kernel scaffold; baseline (speedup 1.0000x reference)
import jax
import jax.numpy as jnp
from jax.experimental import pallas as pl


def kernel(x, edge_index, edge_attr, time_emb, batch, node_w1, node_b1, node_w2, node_b2, edge_w1, edge_b1, edge_w2, edge_b2, time_w, time_b, c1_w1, c1_b1, c1_w2, c1_b2, c2_w1, c2_b1, c2_w2, c2_b2, c3_w1, c3_b1, c3_w2, c3_b2, f_w1, f_b1, f_w2, f_b2):
    raise NotImplementedError("write your pallas kernel here")


# speedup vs baseline: 2.1047x; 2.1047x over previous
import functools

import jax
import jax.numpy as jnp
from jax import lax
from jax.experimental import pallas as pl
from jax.experimental.pallas import tpu as pltpu
from jax.experimental.pallas import tpu_sc as plsc

N = 50000
E = 800000
D = 64
H = 32
G = 64

NC = 2
NS = 16

CB = 256
CHUNKS = 200
E_TILE = CB * CHUNKS
E_PAD = E_TILE * NS
STRIPE = 3136
N_PAD = STRIPE * NS

_f32 = jnp.float32


def _silu(v):
    return v * jax.nn.sigmoid(v)


_NB = 1000


def _encoder_body(x_ref, b_ref, te_ref, tw_ref, tb_ref, w1_ref, b1_ref,
                  w2_ref, b2_ref, o0_ref, o1_ref):
    xv = x_ref[...]
    h1 = _silu(jnp.dot(xv, w1_ref[...], preferred_element_type=_f32)
               + b1_ref[...])
    h2 = jnp.dot(h1, w2_ref[...], preferred_element_type=_f32) + b2_ref[...]
    tproj = (jnp.dot(_silu(te_ref[...]), tw_ref[...],
                     preferred_element_type=_f32) + tb_ref[...])
    onehot = (b_ref[...] == lax.broadcasted_iota(jnp.int32, (_NB, G), 1)
              ).astype(_f32)
    h2 = h2 + jnp.dot(onehot, tproj, preferred_element_type=_f32, precision=lax.Precision.HIGHEST)
    o0_ref[...] = h2[:, :H]
    o1_ref[...] = h2[:, H:]


def _encoder(x, batch2d, time_emb, time_w, time_b, w1, b1, w2, b2):
    full = lambda s: pl.BlockSpec(s, lambda i: (0, 0))
    return pl.pallas_call(
        _encoder_body,
        grid=(N // _NB,),
        in_specs=[
            pl.BlockSpec((_NB, 128), lambda i: (i, 0)),
            pl.BlockSpec((_NB, 1), lambda i: (i, 0)),
            full((G, 32)), full((32, D)), full((1, D)),
            full((128, D)), full((1, D)), full((D, D)), full((1, D)),
        ],
        out_specs=[pl.BlockSpec((_NB, H), lambda i: (i, 0))] * 2,
        out_shape=[jax.ShapeDtypeStruct((N, H), _f32)] * 2,
    )(x, batch2d, time_emb, time_w, time_b, w1, b1, w2, b2)


_EB = 2048


def _edge_body(ea_ref, w1_ref, b1_ref, w2_ref, b2_ref, o0_ref, o1_ref):
    e1 = _silu(jnp.dot(ea_ref[...], w1_ref[...], preferred_element_type=_f32)
               + b1_ref[...])
    e2 = jnp.dot(e1, w2_ref[...], preferred_element_type=_f32) + b2_ref[...]
    o0_ref[...] = e2[:, :H]
    o1_ref[...] = e2[:, H:]


def _edge_encoder(ea_pad, w1, b1, w2, b2):
    full = lambda s: pl.BlockSpec(s, lambda i: (0, 0))
    return pl.pallas_call(
        _edge_body,
        grid=(E_PAD // _EB,),
        in_specs=[
            pl.BlockSpec((_EB, 8), lambda i: (i, 0)),
            full((8, D)), full((1, D)), full((D, D)), full((1, D)),
        ],
        out_specs=[pl.BlockSpec((_EB, H), lambda i: (i, 0))] * 2,
        out_shape=[jax.ShapeDtypeStruct((E_PAD, H), _f32)] * 2,
    )(ea_pad, w1, b1, w2, b2)


def _layer_body(h0_ref, h1_ref, a0_ref, a1_ref, w1_ref, b1_ref, w2_ref,
                b2_ref, o0_ref, o1_ref):
    g0 = h0_ref[...] + a0_ref[...]
    g1 = h1_ref[...] + a1_ref[...]
    u = (jnp.dot(g0, w1_ref[:H, :], preferred_element_type=_f32)
         + jnp.dot(g1, w1_ref[H:, :], preferred_element_type=_f32)
         + b1_ref[...])
    v = jnp.dot(_silu(u), w2_ref[...], preferred_element_type=_f32) + b2_ref[...]
    hn = _silu(v)
    o0_ref[...] = hn[:, :H]
    o1_ref[...] = hn[:, H:]


def _gine_update(h0, h1, a0, a1, w1, b1, w2, b2):
    full = lambda s: pl.BlockSpec(s, lambda i: (0, 0))
    return pl.pallas_call(
        _layer_body,
        grid=(N // _NB,),
        in_specs=[
            pl.BlockSpec((_NB, H), lambda i: (i, 0)),
            pl.BlockSpec((_NB, H), lambda i: (i, 0)),
            pl.BlockSpec((_NB, H), lambda i: (i, 0)),
            pl.BlockSpec((_NB, H), lambda i: (i, 0)),
            full((D, D)), full((1, D)), full((D, D)), full((1, D)),
        ],
        out_specs=[pl.BlockSpec((_NB, H), lambda i: (i, 0))] * 2,
        out_shape=[jax.ShapeDtypeStruct((N, H), _f32)] * 2,
    )(h0, h1, a0, a1, w1, b1, w2, b2)


def _final_body(h0_ref, h1_ref, w1_ref, b1_ref, w2_ref, b2_ref, o_ref):
    u = _silu(jnp.dot(h0_ref[...], w1_ref[:H, :], preferred_element_type=_f32)
              + jnp.dot(h1_ref[...], w1_ref[H:, :], preferred_element_type=_f32)
              + b1_ref[...])
    o_ref[...] = (jnp.dot(u, w2_ref[...], preferred_element_type=_f32)
                  + b2_ref[...])


def _final(h0, h1, w1, b1, w2, b2):
    full = lambda s: pl.BlockSpec(s, lambda i: (0, 0))
    return pl.pallas_call(
        _final_body,
        grid=(N // _NB,),
        in_specs=[
            pl.BlockSpec((_NB, H), lambda i: (i, 0)),
            pl.BlockSpec((_NB, H), lambda i: (i, 0)),
            full((D, D)), full((1, D)), full((D, 1)), full((1, 1)),
        ],
        out_specs=pl.BlockSpec((_NB, 1), lambda i: (i, 0)),
        out_shape=jax.ShapeDtypeStruct((N, 1), _f32),
    )(h0, h1, w1, b1, w2, b2)


_R = CB // 128


def _sc_body(h0_hbm, h1_hbm, ef0_hbm, ef1_hbm, src_hbm, dst_hbm, z_hbm,
             o0_hbm, o1_hbm,
             idx_s, idx_d, hrow, efrow, msg, agg_sh, gsem, esem, ssem):
    c = lax.axis_index("c")
    s = lax.axis_index("s")

    pltpu.sync_copy(z_hbm, agg_sh.at[pl.ds(s * STRIPE, STRIPE)])
    plsc.subcore_barrier()

    def edge_phase(h_hbm, ef_hbm):
        def chunk(g, carry):
            base_row = s * (E_TILE // 128) + g * _R
            ebase = s * E_TILE + g * CB
            pltpu.sync_copy(src_hbm.at[pl.ds(base_row, _R)], idx_s)
            pltpu.sync_copy(dst_hbm.at[pl.ds(base_row, _R)], idx_d)
            gathers = [
                pltpu.async_copy(h_hbm.at[idx_s.at[j]],
                                 hrow.at[pl.ds(j * 128, 128)], gsem)
                for j in range(_R)
            ]
            eload = pltpu.async_copy(ef_hbm.at[pl.ds(ebase, CB)], efrow, esem)
            for d in gathers:
                d.wait()
            eload.wait()

            def cbody(i, carry2):
                a0 = hrow[i, pl.ds(0, 16)] + efrow[i, pl.ds(0, 16)]
                a1 = hrow[i, pl.ds(16, 16)] + efrow[i, pl.ds(16, 16)]
                msg[i, pl.ds(0, 16)] = jnp.maximum(a0, 0.0)
                msg[i, pl.ds(16, 16)] = jnp.maximum(a1, 0.0)
                return carry2
            lax.fori_loop(0, CB, cbody, 0)

            scats = [
                pltpu.async_copy(msg.at[pl.ds(j * 128, 128)],
                                 agg_sh.at[idx_d.at[j]], ssem, add=True)
                for j in range(_R)
            ]
            for d in scats:
                d.wait()
            return carry
        lax.fori_loop(0, CHUNKS, chunk, 0)

    @pl.when(c == 0)
    def _():
        edge_phase(h0_hbm, ef0_hbm)

    @pl.when(c == 1)
    def _():
        edge_phase(h1_hbm, ef1_hbm)

    plsc.subcore_barrier()

    @pl.when(c == 0)
    def _():
        pltpu.sync_copy(agg_sh.at[pl.ds(s * STRIPE, STRIPE)],
                        o0_hbm.at[pl.ds(s * STRIPE, STRIPE)])

    @pl.when(c == 1)
    def _():
        pltpu.sync_copy(agg_sh.at[pl.ds(s * STRIPE, STRIPE)],
                        o1_hbm.at[pl.ds(s * STRIPE, STRIPE)])


@functools.lru_cache(maxsize=1)
def _sc_aggregate_fn():
    mesh = plsc.VectorSubcoreMesh(
        core_axis_name="c", subcore_axis_name="s",
        num_cores=NC, num_subcores=NS)
    return pl.kernel(
        _sc_body,
        out_type=[jax.ShapeDtypeStruct((N_PAD, H), _f32)] * 2,
        mesh=mesh,
        compiler_params=pltpu.CompilerParams(use_tc_tiling_on_sc=False),
        scratch_types=[
            pltpu.VMEM((_R, 128), jnp.int32),
            pltpu.VMEM((_R, 128), jnp.int32),
            pltpu.VMEM((CB, H), _f32),
            pltpu.VMEM((CB, H), _f32),
            pltpu.VMEM((CB, H), _f32),
            pltpu.VMEM_SHARED((N_PAD, H), _f32),
            pltpu.SemaphoreType.DMA,
            pltpu.SemaphoreType.DMA,
            pltpu.SemaphoreType.DMA,
        ],
    )


def _sc_aggregate(*args):
    return _sc_aggregate_fn()(*args)


def kernel(x, edge_index, edge_attr, time_emb, batch,
           node_w1, node_b1, node_w2, node_b2,
           edge_w1, edge_b1, edge_w2, edge_b2,
           time_w, time_b,
           c1_w1, c1_b1, c1_w2, c1_b2,
           c2_w1, c2_b1, c2_w2, c2_b2,
           c3_w1, c3_b1, c3_w2, c3_b2,
           f_w1, f_b1, f_w2, f_b2):
    pad = E_PAD - E
    src = edge_index[0].astype(jnp.int32)
    dst = edge_index[1].astype(jnp.int32)
    src2d = jnp.concatenate([src, jnp.zeros((pad,), jnp.int32)]
                            ).reshape(E_PAD // 128, 128)
    dst2d = jnp.concatenate([dst, jnp.full((pad,), N, jnp.int32)]
                            ).reshape(E_PAD // 128, 128)
    ea_pad = jnp.zeros((E_PAD, 8), _f32).at[:E, :4].set(edge_attr)
    ew1_pad = jnp.zeros((8, D), _f32).at[:4, :].set(edge_w1)
    batch2d = batch.astype(jnp.int32).reshape(N, 1)
    zeros_stripe = jnp.zeros((STRIPE, H), _f32)

    r2 = lambda b: b.reshape(1, -1)

    h0, h1 = _encoder(x, batch2d, time_emb, time_w, r2(time_b),
                      node_w1, r2(node_b1), node_w2, r2(node_b2))
    ef0, ef1 = _edge_encoder(ea_pad, ew1_pad, r2(edge_b1), edge_w2,
                             r2(edge_b2))

    for (w1, b1, w2, b2) in ((c1_w1, c1_b1, c1_w2, c1_b2),
                             (c2_w1, c2_b1, c2_w2, c2_b2),
                             (c3_w1, c3_b1, c3_w2, c3_b2)):
        a0, a1 = _sc_aggregate(h0, h1, ef0, ef1, src2d, dst2d, zeros_stripe)
        h0, h1 = _gine_update(h0, h1, a0[:N], a1[:N], w1, r2(b1), w2, r2(b2))

    return _final(h0, h1, f_w1, r2(f_b1), f_w2, f_b2.reshape(1, 1))


# speedup vs baseline: 2.3913x; 1.1361x over previous
import functools

import jax
import jax.numpy as jnp
from jax import lax
from jax.experimental import pallas as pl
from jax.experimental.pallas import tpu as pltpu
from jax.experimental.pallas import tpu_sc as plsc

N = 50000
E = 800000
D = 64
H = 32
G = 64

NC = 2
NS = 16

CB = 256
CHUNKS = 200
E_TILE = CB * CHUNKS
E_PAD = E_TILE * NS
STRIPE = 3136
N_PAD = STRIPE * NS

_f32 = jnp.float32


def _silu(v):
    return v * jax.nn.sigmoid(v)


_NB = 1000


def _encoder_body(x_ref, b_ref, te_ref, tw_ref, tb_ref, w1_ref, b1_ref,
                  w2_ref, b2_ref, o0_ref, o1_ref):
    xv = x_ref[...]
    h1 = _silu(jnp.dot(xv, w1_ref[...], preferred_element_type=_f32)
               + b1_ref[...])
    h2 = jnp.dot(h1, w2_ref[...], preferred_element_type=_f32) + b2_ref[...]
    tproj = (jnp.dot(_silu(te_ref[...]), tw_ref[...],
                     preferred_element_type=_f32) + tb_ref[...])
    onehot = (b_ref[...] == lax.broadcasted_iota(jnp.int32, (_NB, G), 1)
              ).astype(_f32)
    h2 = h2 + jnp.dot(onehot, tproj, preferred_element_type=_f32, precision=lax.Precision.HIGHEST)
    o0_ref[...] = h2[:, :H]
    o1_ref[...] = h2[:, H:]


def _encoder(x, batch2d, time_emb, time_w, time_b, w1, b1, w2, b2):
    full = lambda s: pl.BlockSpec(s, lambda i: (0, 0))
    return pl.pallas_call(
        _encoder_body,
        grid=(N // _NB,),
        in_specs=[
            pl.BlockSpec((_NB, 128), lambda i: (i, 0)),
            pl.BlockSpec((_NB, 1), lambda i: (i, 0)),
            full((G, 32)), full((32, D)), full((1, D)),
            full((128, D)), full((1, D)), full((D, D)), full((1, D)),
        ],
        out_specs=[pl.BlockSpec((_NB, H), lambda i: (i, 0))] * 2,
        out_shape=[jax.ShapeDtypeStruct((N, H), _f32)] * 2,
    )(x, batch2d, time_emb, time_w, time_b, w1, b1, w2, b2)


_EB = 2048


def _edge_body(ea_ref, w1_ref, b1_ref, w2_ref, b2_ref, o0_ref, o1_ref):
    e1 = _silu(jnp.dot(ea_ref[...], w1_ref[...], preferred_element_type=_f32)
               + b1_ref[...])
    e2 = jnp.dot(e1, w2_ref[...], preferred_element_type=_f32) + b2_ref[...]
    o0_ref[...] = e2[:, :H]
    o1_ref[...] = e2[:, H:]


def _edge_encoder(ea, w1, b1, w2, b2):
    full = lambda s: pl.BlockSpec(s, lambda i: (0, 0))
    return pl.pallas_call(
        _edge_body,
        grid=(pl.cdiv(E, _EB),),
        in_specs=[
            pl.BlockSpec((_EB, 4), lambda i: (i, 0)),
            full((4, D)), full((1, D)), full((D, D)), full((1, D)),
        ],
        out_specs=[pl.BlockSpec((_EB, H), lambda i: (i, 0))] * 2,
        out_shape=[jax.ShapeDtypeStruct((E_PAD, H), _f32)] * 2,
    )(ea, w1, b1, w2, b2)


def _layer_body(h0_ref, h1_ref, a0_ref, a1_ref, w1_ref, b1_ref, w2_ref,
                b2_ref, o0_ref, o1_ref):
    g0 = h0_ref[...] + a0_ref[...]
    g1 = h1_ref[...] + a1_ref[...]
    u = (jnp.dot(g0, w1_ref[:H, :], preferred_element_type=_f32)
         + jnp.dot(g1, w1_ref[H:, :], preferred_element_type=_f32)
         + b1_ref[...])
    v = jnp.dot(_silu(u), w2_ref[...], preferred_element_type=_f32) + b2_ref[...]
    hn = _silu(v)
    o0_ref[...] = hn[:, :H]
    o1_ref[...] = hn[:, H:]


def _gine_update(h0, h1, a0, a1, w1, b1, w2, b2):
    full = lambda s: pl.BlockSpec(s, lambda i: (0, 0))
    return pl.pallas_call(
        _layer_body,
        grid=(N // _NB,),
        in_specs=[
            pl.BlockSpec((_NB, H), lambda i: (i, 0)),
            pl.BlockSpec((_NB, H), lambda i: (i, 0)),
            pl.BlockSpec((_NB, H), lambda i: (i, 0)),
            pl.BlockSpec((_NB, H), lambda i: (i, 0)),
            full((D, D)), full((1, D)), full((D, D)), full((1, D)),
        ],
        out_specs=[pl.BlockSpec((_NB, H), lambda i: (i, 0))] * 2,
        out_shape=[jax.ShapeDtypeStruct((N, H), _f32)] * 2,
    )(h0, h1, a0, a1, w1, b1, w2, b2)


def _final_body(h0_ref, h1_ref, w1_ref, b1_ref, w2_ref, b2_ref, o_ref):
    u = _silu(jnp.dot(h0_ref[...], w1_ref[:H, :], preferred_element_type=_f32)
              + jnp.dot(h1_ref[...], w1_ref[H:, :], preferred_element_type=_f32)
              + b1_ref[...])
    o_ref[...] = (jnp.dot(u, w2_ref[...], preferred_element_type=_f32)
                  + b2_ref[...])


def _final(h0, h1, w1, b1, w2, b2):
    full = lambda s: pl.BlockSpec(s, lambda i: (0, 0))
    return pl.pallas_call(
        _final_body,
        grid=(N // _NB,),
        in_specs=[
            pl.BlockSpec((_NB, H), lambda i: (i, 0)),
            pl.BlockSpec((_NB, H), lambda i: (i, 0)),
            full((D, D)), full((1, D)), full((D, 1)), full((1, 1)),
        ],
        out_specs=pl.BlockSpec((_NB, 1), lambda i: (i, 0)),
        out_shape=jax.ShapeDtypeStruct((N, 1), _f32),
    )(h0, h1, w1, b1, w2, b2)


_R = CB // 128


def _sc_body(h0_hbm, h1_hbm, ef0_hbm, ef1_hbm, src_hbm, dst_hbm, z_hbm,
             o0_hbm, o1_hbm,
             idx_s, idx_d, hrow, efrow, msg, agg_sh, gsem, esem, ssem):
    c = lax.axis_index("c")
    s = lax.axis_index("s")

    pltpu.sync_copy(z_hbm, agg_sh.at[pl.ds(s * STRIPE, STRIPE)])
    plsc.subcore_barrier()

    def edge_phase(h_hbm, ef_hbm):
        def chunk(g, carry):
            base_row = s * (E_TILE // 128) + g * _R
            ebase = s * E_TILE + g * CB
            pltpu.sync_copy(src_hbm.at[pl.ds(base_row, _R)], idx_s)
            pltpu.sync_copy(dst_hbm.at[pl.ds(base_row, _R)], idx_d)
            gathers = [
                pltpu.async_copy(h_hbm.at[idx_s.at[j]],
                                 hrow.at[pl.ds(j * 128, 128)], gsem)
                for j in range(_R)
            ]
            eload = pltpu.async_copy(ef_hbm.at[pl.ds(ebase, CB)], efrow, esem)
            for d in gathers:
                d.wait()
            eload.wait()

            def cbody(i, carry2):
                a0 = hrow[i, pl.ds(0, 16)] + efrow[i, pl.ds(0, 16)]
                a1 = hrow[i, pl.ds(16, 16)] + efrow[i, pl.ds(16, 16)]
                msg[i, pl.ds(0, 16)] = jnp.maximum(a0, 0.0)
                msg[i, pl.ds(16, 16)] = jnp.maximum(a1, 0.0)
                return carry2
            lax.fori_loop(0, CB, cbody, 0)

            scats = [
                pltpu.async_copy(msg.at[pl.ds(j * 128, 128)],
                                 agg_sh.at[idx_d.at[j]], ssem, add=True)
                for j in range(_R)
            ]
            for d in scats:
                d.wait()
            return carry
        lax.fori_loop(0, CHUNKS, chunk, 0)

    @pl.when(c == 0)
    def _():
        edge_phase(h0_hbm, ef0_hbm)

    @pl.when(c == 1)
    def _():
        edge_phase(h1_hbm, ef1_hbm)

    plsc.subcore_barrier()

    @pl.when(c == 0)
    def _():
        pltpu.sync_copy(agg_sh.at[pl.ds(s * STRIPE, STRIPE)],
                        o0_hbm.at[pl.ds(s * STRIPE, STRIPE)])

    @pl.when(c == 1)
    def _():
        pltpu.sync_copy(agg_sh.at[pl.ds(s * STRIPE, STRIPE)],
                        o1_hbm.at[pl.ds(s * STRIPE, STRIPE)])


@functools.lru_cache(maxsize=1)
def _sc_aggregate_fn():
    mesh = plsc.VectorSubcoreMesh(
        core_axis_name="c", subcore_axis_name="s",
        num_cores=NC, num_subcores=NS)
    return pl.kernel(
        _sc_body,
        out_type=[jax.ShapeDtypeStruct((N_PAD, H), _f32)] * 2,
        mesh=mesh,
        compiler_params=pltpu.CompilerParams(use_tc_tiling_on_sc=False),
        scratch_types=[
            pltpu.VMEM((_R, 128), jnp.int32),
            pltpu.VMEM((_R, 128), jnp.int32),
            pltpu.VMEM((CB, H), _f32),
            pltpu.VMEM((CB, H), _f32),
            pltpu.VMEM((CB, H), _f32),
            pltpu.VMEM_SHARED((N_PAD, H), _f32),
            pltpu.SemaphoreType.DMA,
            pltpu.SemaphoreType.DMA,
            pltpu.SemaphoreType.DMA,
        ],
    )


def _sc_aggregate(*args):
    return _sc_aggregate_fn()(*args)


def kernel(x, edge_index, edge_attr, time_emb, batch,
           node_w1, node_b1, node_w2, node_b2,
           edge_w1, edge_b1, edge_w2, edge_b2,
           time_w, time_b,
           c1_w1, c1_b1, c1_w2, c1_b2,
           c2_w1, c2_b1, c2_w2, c2_b2,
           c3_w1, c3_b1, c3_w2, c3_b2,
           f_w1, f_b1, f_w2, f_b2):
    pad = E_PAD - E
    src = edge_index[0].astype(jnp.int32)
    dst = edge_index[1].astype(jnp.int32)
    src2d = jnp.concatenate([src, jnp.zeros((pad,), jnp.int32)]
                            ).reshape(E_PAD // 128, 128)
    dst2d = jnp.concatenate([dst, jnp.full((pad,), N, jnp.int32)]
                            ).reshape(E_PAD // 128, 128)
    batch2d = batch.astype(jnp.int32).reshape(N, 1)
    zeros_stripe = jnp.zeros((STRIPE, H), _f32)

    r2 = lambda b: b.reshape(1, -1)

    h0, h1 = _encoder(x, batch2d, time_emb, time_w, r2(time_b),
                      node_w1, r2(node_b1), node_w2, r2(node_b2))
    ef0, ef1 = _edge_encoder(edge_attr, edge_w1, r2(edge_b1), edge_w2,
                             r2(edge_b2))

    for (w1, b1, w2, b2) in ((c1_w1, c1_b1, c1_w2, c1_b2),
                             (c2_w1, c2_b1, c2_w2, c2_b2),
                             (c3_w1, c3_b1, c3_w2, c3_b2)):
        a0, a1 = _sc_aggregate(h0, h1, ef0, ef1, src2d, dst2d, zeros_stripe)
        h0, h1 = _gine_update(h0, h1, a0, a1, w1, r2(b1), w2, r2(b2))

    return _final(h0, h1, f_w1, r2(f_b1), f_w2, f_b2.reshape(1, 1))


# speedup vs baseline: 3.2047x; 1.3402x over previous
import functools

import jax
import jax.numpy as jnp
from jax import lax
from jax.experimental import pallas as pl
from jax.experimental.pallas import tpu as pltpu
from jax.experimental.pallas import tpu_sc as plsc

N = 50000
E = 800000
D = 64
H = 32
G = 64

NC = 2
NS = 16

CB = 128
CHUNKS = 400
E_TILE = CB * CHUNKS
E_PAD = E_TILE * NS
STRIPE = 3136
N_PAD = STRIPE * NS

_f32 = jnp.float32


def _silu(v):
    return v * jax.nn.sigmoid(v)


_NB = 1000


def _encoder_body(x_ref, b_ref, te_ref, tw_ref, tb_ref, w1_ref, b1_ref,
                  w2_ref, b2_ref, o0_ref, o1_ref):
    xv = x_ref[...]
    h1 = _silu(jnp.dot(xv, w1_ref[...], preferred_element_type=_f32)
               + b1_ref[...])
    h2 = jnp.dot(h1, w2_ref[...], preferred_element_type=_f32) + b2_ref[...]
    tproj = (jnp.dot(_silu(te_ref[...]), tw_ref[...],
                     preferred_element_type=_f32) + tb_ref[...])
    onehot = (b_ref[...] == lax.broadcasted_iota(jnp.int32, (_NB, G), 1)
              ).astype(_f32)
    h2 = h2 + jnp.dot(onehot, tproj, preferred_element_type=_f32, precision=lax.Precision.HIGHEST)
    o0_ref[...] = h2[:, :H]
    o1_ref[...] = h2[:, H:]


def _encoder(x, batch2d, time_emb, time_w, time_b, w1, b1, w2, b2):
    full = lambda s: pl.BlockSpec(s, lambda i: (0, 0))
    return pl.pallas_call(
        _encoder_body,
        grid=(N // _NB,),
        in_specs=[
            pl.BlockSpec((_NB, 128), lambda i: (i, 0)),
            pl.BlockSpec((_NB, 1), lambda i: (i, 0)),
            full((G, 32)), full((32, D)), full((1, D)),
            full((128, D)), full((1, D)), full((D, D)), full((1, D)),
        ],
        out_specs=[pl.BlockSpec((_NB, H), lambda i: (i, 0))] * 2,
        out_shape=[jax.ShapeDtypeStruct((N, H), _f32)] * 2,
    )(x, batch2d, time_emb, time_w, time_b, w1, b1, w2, b2)


_EB4 = 4096


def _edge_body(ea_ref, w1_ref, b1_ref, w2_ref, b2_ref, o0_ref, o1_ref):
    p1 = _silu(jnp.dot(ea_ref[...], w1_ref[...], preferred_element_type=_f32)
               + b1_ref[...])
    p2 = jnp.dot(p1, w2_ref[...], preferred_element_type=_f32) + b2_ref[...]
    o0_ref[...] = jnp.concatenate([p2[:, 64 * q:64 * q + H] for q in range(4)],
                                  axis=1)
    o1_ref[...] = jnp.concatenate(
        [p2[:, 64 * q + H:64 * q + D] for q in range(4)], axis=1)


def _edge_encoder(ea4, w1p, b1p, w2p, b2p):
    full = lambda s: pl.BlockSpec(s, lambda i: (0, 0))
    return pl.pallas_call(
        _edge_body,
        grid=(pl.cdiv(E // 4, _EB4),),
        in_specs=[
            pl.BlockSpec((_EB4, 16), lambda i: (i, 0)),
            full((16, 256)), full((1, 256)), full((256, 256)), full((1, 256)),
        ],
        out_specs=[pl.BlockSpec((_EB4, 128), lambda i: (i, 0))] * 2,
        out_shape=[jax.ShapeDtypeStruct((E_PAD // 4, 128), _f32)] * 2,
    )(ea4, w1p, b1p, w2p, b2p)


def _layer_body(h0_ref, h1_ref, a0_ref, a1_ref, w1_ref, b1_ref, w2_ref,
                b2_ref, o0_ref, o1_ref):
    g0 = h0_ref[...] + a0_ref[...]
    g1 = h1_ref[...] + a1_ref[...]
    u = (jnp.dot(g0, w1_ref[:H, :], preferred_element_type=_f32)
         + jnp.dot(g1, w1_ref[H:, :], preferred_element_type=_f32)
         + b1_ref[...])
    v = jnp.dot(_silu(u), w2_ref[...], preferred_element_type=_f32) + b2_ref[...]
    hn = _silu(v)
    o0_ref[...] = hn[:, :H]
    o1_ref[...] = hn[:, H:]


def _gine_update(h0, h1, a0, a1, w1, b1, w2, b2):
    full = lambda s: pl.BlockSpec(s, lambda i: (0, 0))
    return pl.pallas_call(
        _layer_body,
        grid=(N // _NB,),
        in_specs=[
            pl.BlockSpec((_NB, H), lambda i: (i, 0)),
            pl.BlockSpec((_NB, H), lambda i: (i, 0)),
            pl.BlockSpec((_NB, H), lambda i: (i, 0)),
            pl.BlockSpec((_NB, H), lambda i: (i, 0)),
            full((D, D)), full((1, D)), full((D, D)), full((1, D)),
        ],
        out_specs=[pl.BlockSpec((_NB, H), lambda i: (i, 0))] * 2,
        out_shape=[jax.ShapeDtypeStruct((N, H), _f32)] * 2,
    )(h0, h1, a0, a1, w1, b1, w2, b2)


def _final_body(h0_ref, h1_ref, w1_ref, b1_ref, w2_ref, b2_ref, o_ref):
    u = _silu(jnp.dot(h0_ref[...], w1_ref[:H, :], preferred_element_type=_f32)
              + jnp.dot(h1_ref[...], w1_ref[H:, :], preferred_element_type=_f32)
              + b1_ref[...])
    o_ref[...] = (jnp.dot(u, w2_ref[...], preferred_element_type=_f32)
                  + b2_ref[...])


def _final(h0, h1, w1, b1, w2, b2):
    full = lambda s: pl.BlockSpec(s, lambda i: (0, 0))
    return pl.pallas_call(
        _final_body,
        grid=(N // _NB,),
        in_specs=[
            pl.BlockSpec((_NB, H), lambda i: (i, 0)),
            pl.BlockSpec((_NB, H), lambda i: (i, 0)),
            full((D, D)), full((1, D)), full((D, 1)), full((1, 1)),
        ],
        out_specs=pl.BlockSpec((_NB, 1), lambda i: (i, 0)),
        out_shape=jax.ShapeDtypeStruct((N, 1), _f32),
    )(h0, h1, w1, b1, w2, b2)


_GRP = 8
_BODY = 2 * _GRP
_KOUT = CHUNKS // _BODY


def _sc_body(h0_hbm, h1_hbm, ef0_hbm, ef1_hbm, sd_hbm, z_hbm,
             o0_hbm, o1_hbm,
             bufI0, bufI1, hrow0, hrow1, efb0, efb1, msg0, msg1, agg_sh,
             isem0, isem1, gsem0, gsem1, esem0, esem1, ssem0, ssem1):
    c = lax.axis_index("c")
    s = lax.axis_index("s")

    bufI = (bufI0, bufI1)
    hrow = (hrow0, hrow1)
    efb = (efb0, efb1)
    msg = (msg0, msg1)
    isem = (isem0, isem1)
    gsem = (gsem0, gsem1)
    esem = (esem0, esem1)
    ssem = (ssem0, ssem1)

    pltpu.sync_copy(z_hbm, agg_sh.at[pl.ds(s * STRIPE, STRIPE)])
    plsc.subcore_barrier()

    sd_base = 2 * s * CHUNKS
    ef_base = s * (E_TILE // 4)

    def edge_phase(h_hbm, ef_hbm):
        def idx_row(m, which):
            return bufI[(m // _GRP) % 2].at[2 * (m % _GRP) + which]

        def issue_data(k, m):
            b = m % 2
            g = 16 * k + m
            gd = pltpu.async_copy(h_hbm.at[idx_row(m, 0)], hrow[b], gsem[b])
            ed = pltpu.async_copy(
                ef_hbm.at[pl.ds(ef_base + g * (CB // 4), CB // 4)],
                efb[b], esem[b])
            return gd, ed

        def wait_data(b):
            pltpu.make_async_copy(h_hbm.at[bufI[0].at[0]], hrow[b],
                                  gsem[b]).wait()
            pltpu.make_async_copy(ef_hbm.at[pl.ds(0, CB // 4)], efb[b],
                                  esem[b]).wait()

        def wait_scatter(b):
            pltpu.make_async_copy(msg[b], agg_sh.at[bufI[0].at[1]],
                                  ssem[b]).wait()

        def compute(b):
            @plsc.parallel_loop(0, CB // 4, 1, unroll=2)
            def _(rr):
                for q in range(4):
                    i = 4 * rr + q
                    lo = hrow[b][i, pl.ds(0, 16)] + efb[b][rr, pl.ds(32 * q, 16)]
                    hi = (hrow[b][i, pl.ds(16, 16)]
                          + efb[b][rr, pl.ds(32 * q + 16, 16)])
                    msg[b][i, pl.ds(0, 16)] = jnp.maximum(lo, 0.0)
                    msg[b][i, pl.ds(16, 16)] = jnp.maximum(hi, 0.0)

        pltpu.sync_copy(sd_hbm.at[pl.ds(sd_base, 2 * _GRP)], bufI[0])
        issue_data(0, 0)

        def outer(k, carry):
            for m in range(_BODY):
                b = m % 2
                wait_data(b)
                if m >= 2:
                    wait_scatter(b)
                else:
                    @pl.when(k > 0)
                    def _():
                        wait_scatter(b)
                if m == 2:
                    pltpu.async_copy(
                        sd_hbm.at[pl.ds(sd_base + (32 * k + 16), 2 * _GRP)],
                        bufI[1], isem[1])
                if m == 10:
                    @pl.when(k < _KOUT - 1)
                    def _():
                        pltpu.async_copy(
                            sd_hbm.at[pl.ds(sd_base + (32 * k + 32), 2 * _GRP)],
                            bufI[0], isem[0])
                if m == _GRP - 1:
                    pltpu.make_async_copy(sd_hbm.at[pl.ds(0, 2 * _GRP)],
                                          bufI[1], isem[1]).wait()
                if m == _BODY - 1:
                    @pl.when(k < _KOUT - 1)
                    def _():
                        pltpu.make_async_copy(sd_hbm.at[pl.ds(0, 2 * _GRP)],
                                              bufI[0], isem[0]).wait()
                        issue_data(k + 1, 0)
                else:
                    issue_data(k, m + 1)
                compute(b)
                pltpu.async_copy(msg[b], agg_sh.at[idx_row(m, 1)], ssem[b],
                                 add=True)
            return carry

        lax.fori_loop(0, _KOUT, outer, 0)
        wait_scatter(0)
        wait_scatter(1)

    @pl.when(c == 0)
    def _():
        edge_phase(h0_hbm, ef0_hbm)

    @pl.when(c == 1)
    def _():
        edge_phase(h1_hbm, ef1_hbm)

    plsc.subcore_barrier()

    @pl.when(c == 0)
    def _():
        pltpu.sync_copy(agg_sh.at[pl.ds(s * STRIPE, STRIPE)],
                        o0_hbm.at[pl.ds(s * STRIPE, STRIPE)])

    @pl.when(c == 1)
    def _():
        pltpu.sync_copy(agg_sh.at[pl.ds(s * STRIPE, STRIPE)],
                        o1_hbm.at[pl.ds(s * STRIPE, STRIPE)])


@functools.lru_cache(maxsize=1)
def _sc_aggregate_fn():
    mesh = plsc.VectorSubcoreMesh(
        core_axis_name="c", subcore_axis_name="s",
        num_cores=NC, num_subcores=NS)
    return pl.kernel(
        _sc_body,
        out_type=[jax.ShapeDtypeStruct((N_PAD, H), _f32)] * 2,
        mesh=mesh,
        compiler_params=pltpu.CompilerParams(use_tc_tiling_on_sc=False),
        scratch_types=[
            pltpu.VMEM((2 * _GRP, 128), jnp.int32),
            pltpu.VMEM((2 * _GRP, 128), jnp.int32),
            pltpu.VMEM((CB, H), _f32),
            pltpu.VMEM((CB, H), _f32),
            pltpu.VMEM((CB // 4, 128), _f32),
            pltpu.VMEM((CB // 4, 128), _f32),
            pltpu.VMEM((CB, H), _f32),
            pltpu.VMEM((CB, H), _f32),
            pltpu.VMEM_SHARED((N_PAD, H), _f32),
        ] + [pltpu.SemaphoreType.DMA] * 8,
    )


def _sc_aggregate(*args):
    return _sc_aggregate_fn()(*args)


def kernel(x, edge_index, edge_attr, time_emb, batch,
           node_w1, node_b1, node_w2, node_b2,
           edge_w1, edge_b1, edge_w2, edge_b2,
           time_w, time_b,
           c1_w1, c1_b1, c1_w2, c1_b2,
           c2_w1, c2_b1, c2_w2, c2_b2,
           c3_w1, c3_b1, c3_w2, c3_b2,
           f_w1, f_b1, f_w2, f_b2):
    pad = E_PAD - E
    src = edge_index[0].astype(jnp.int32)
    dst = edge_index[1].astype(jnp.int32)
    src2d = jnp.concatenate([src, jnp.zeros((pad,), jnp.int32)]
                            ).reshape(E_PAD // 128, 128)
    dst2d = jnp.concatenate([dst, jnp.full((pad,), N, jnp.int32)]
                            ).reshape(E_PAD // 128, 128)
    sd = jnp.stack([src2d, dst2d], axis=1).reshape(2 * E_PAD // 128, 128)
    batch2d = batch.astype(jnp.int32).reshape(N, 1)
    zeros_stripe = jnp.zeros((STRIPE, H), _f32)

    ea4 = edge_attr.reshape(E // 4, 16)
    w1p = jnp.zeros((16, 256), _f32)
    w2p = jnp.zeros((256, 256), _f32)
    for q in range(4):
        w1p = w1p.at[4 * q:4 * q + 4, 64 * q:64 * q + 64].set(edge_w1)
        w2p = w2p.at[64 * q:64 * q + 64, 64 * q:64 * q + 64].set(edge_w2)
    b1p = jnp.tile(edge_b1, 4).reshape(1, 256)
    b2p = jnp.tile(edge_b2, 4).reshape(1, 256)

    r2 = lambda b: b.reshape(1, -1)

    h0, h1 = _encoder(x, batch2d, time_emb, time_w, r2(time_b),
                      node_w1, r2(node_b1), node_w2, r2(node_b2))
    ef0, ef1 = _edge_encoder(ea4, w1p, b1p, w2p, b2p)

    for (w1, b1, w2, b2) in ((c1_w1, c1_b1, c1_w2, c1_b2),
                             (c2_w1, c2_b1, c2_w2, c2_b2),
                             (c3_w1, c3_b1, c3_w2, c3_b2)):
        a0, a1 = _sc_aggregate(h0, h1, ef0, ef1, sd, zeros_stripe)
        h0, h1 = _gine_update(h0, h1, a0, a1, w1, r2(b1), w2, r2(b2))

    return _final(h0, h1, f_w1, r2(f_b1), f_w2, f_b2.reshape(1, 1))


# speedup vs baseline: 3.2106x; 1.0018x over previous
import functools

import jax
import jax.numpy as jnp
from jax import lax
from jax.experimental import pallas as pl
from jax.experimental.pallas import tpu as pltpu
from jax.experimental.pallas import tpu_sc as plsc

N = 50000
E = 800000
D = 64
H = 32
G = 64

NC = 2
NS = 16

CB = 128
CHUNKS = 400
E_TILE = CB * CHUNKS
E_PAD = E_TILE * NS
STRIPE = 3136
N_PAD = STRIPE * NS

_f32 = jnp.float32


def _silu(v):
    return v * jax.nn.sigmoid(v)


_NB = 1000


def _encoder_body(x_ref, b_ref, te_ref, tw_ref, tb_ref, w1_ref, b1_ref,
                  w2_ref, b2_ref, o0_ref, o1_ref):
    xv = x_ref[...]
    h1 = _silu(jnp.dot(xv, w1_ref[...], preferred_element_type=_f32)
               + b1_ref[...])
    h2 = jnp.dot(h1, w2_ref[...], preferred_element_type=_f32) + b2_ref[...]
    tproj = (jnp.dot(_silu(te_ref[...]), tw_ref[...],
                     preferred_element_type=_f32) + tb_ref[...])
    onehot = (b_ref[...] == lax.broadcasted_iota(jnp.int32, (_NB, G), 1)
              ).astype(_f32)
    h2 = h2 + jnp.dot(onehot, tproj, preferred_element_type=_f32, precision=lax.Precision.HIGHEST)
    o0_ref[...] = h2[:, :H]
    o1_ref[...] = h2[:, H:]


def _encoder(x, batch2d, time_emb, time_w, time_b, w1, b1, w2, b2):
    full = lambda s: pl.BlockSpec(s, lambda i: (0, 0))
    return pl.pallas_call(
        _encoder_body,
        grid=(N // _NB,),
        in_specs=[
            pl.BlockSpec((_NB, 128), lambda i: (i, 0)),
            pl.BlockSpec((_NB, 1), lambda i: (i, 0)),
            full((G, 32)), full((32, D)), full((1, D)),
            full((128, D)), full((1, D)), full((D, D)), full((1, D)),
        ],
        out_specs=[pl.BlockSpec((_NB, H), lambda i: (i, 0))] * 2,
        out_shape=[jax.ShapeDtypeStruct((N, H), _f32)] * 2,
    )(x, batch2d, time_emb, time_w, time_b, w1, b1, w2, b2)


_EB4 = 4096


def _edge_body(ea_ref, w1_ref, b1_ref, w2_ref, b2_ref, o0_ref, o1_ref):
    p1 = _silu(jnp.dot(ea_ref[...], w1_ref[...], preferred_element_type=_f32)
               + b1_ref[...])
    p2 = jnp.dot(p1, w2_ref[...], preferred_element_type=_f32) + b2_ref[...]
    o0_ref[...] = jnp.concatenate([p2[:, 64 * q:64 * q + H] for q in range(4)],
                                  axis=1)
    o1_ref[...] = jnp.concatenate(
        [p2[:, 64 * q + H:64 * q + D] for q in range(4)], axis=1)


def _edge_encoder(ea4, w1p, b1p, w2p, b2p):
    full = lambda s: pl.BlockSpec(s, lambda i: (0, 0))
    return pl.pallas_call(
        _edge_body,
        grid=(pl.cdiv(E // 4, _EB4),),
        in_specs=[
            pl.BlockSpec((_EB4, 16), lambda i: (i, 0)),
            full((16, 256)), full((1, 256)), full((256, 256)), full((1, 256)),
        ],
        out_specs=[pl.BlockSpec((_EB4, 128), lambda i: (i, 0))] * 2,
        out_shape=[jax.ShapeDtypeStruct((E_PAD // 4, 128), _f32)] * 2,
    )(ea4, w1p, b1p, w2p, b2p)


def _layer_body(h0_ref, h1_ref, a0_ref, a1_ref, w1_ref, b1_ref, w2_ref,
                b2_ref, o0_ref, o1_ref):
    g = jnp.concatenate([h0_ref[...] + a0_ref[...],
                         h1_ref[...] + a1_ref[...]], axis=1)
    u = jnp.dot(g, w1_ref[...], preferred_element_type=_f32) + b1_ref[...]
    v = jnp.dot(_silu(u), w2_ref[...], preferred_element_type=_f32) + b2_ref[...]
    hn = _silu(v)
    o0_ref[...] = hn[:, :H]
    o1_ref[...] = hn[:, H:]


def _gine_update(h0, h1, a0, a1, w1, b1, w2, b2):
    full = lambda s: pl.BlockSpec(s, lambda i: (0, 0))
    return pl.pallas_call(
        _layer_body,
        grid=(N // _NB,),
        in_specs=[
            pl.BlockSpec((_NB, H), lambda i: (i, 0)),
            pl.BlockSpec((_NB, H), lambda i: (i, 0)),
            pl.BlockSpec((_NB, H), lambda i: (i, 0)),
            pl.BlockSpec((_NB, H), lambda i: (i, 0)),
            full((D, D)), full((1, D)), full((D, D)), full((1, D)),
        ],
        out_specs=[pl.BlockSpec((_NB, H), lambda i: (i, 0))] * 2,
        out_shape=[jax.ShapeDtypeStruct((N, H), _f32)] * 2,
    )(h0, h1, a0, a1, w1, b1, w2, b2)


def _final_body(h0_ref, h1_ref, w1_ref, b1_ref, w2_ref, b2_ref, o_ref):
    hh = jnp.concatenate([h0_ref[...], h1_ref[...]], axis=1)
    u = _silu(jnp.dot(hh, w1_ref[...], preferred_element_type=_f32)
              + b1_ref[...])
    o_ref[...] = (jnp.dot(u, w2_ref[...], preferred_element_type=_f32)
                  + b2_ref[...])


def _final(h0, h1, w1, b1, w2, b2):
    full = lambda s: pl.BlockSpec(s, lambda i: (0, 0))
    return pl.pallas_call(
        _final_body,
        grid=(N // _NB,),
        in_specs=[
            pl.BlockSpec((_NB, H), lambda i: (i, 0)),
            pl.BlockSpec((_NB, H), lambda i: (i, 0)),
            full((D, D)), full((1, D)), full((D, 1)), full((1, 1)),
        ],
        out_specs=pl.BlockSpec((_NB, 1), lambda i: (i, 0)),
        out_shape=jax.ShapeDtypeStruct((N, 1), _f32),
    )(h0, h1, w1, b1, w2, b2)


_GRP = 8
_BODY = 2 * _GRP
_KOUT = CHUNKS // _BODY


def _sc_body(h0_hbm, h1_hbm, ef0_hbm, ef1_hbm, sd_hbm, z_hbm,
             o0_hbm, o1_hbm,
             bufI0, bufI1, hrow0, hrow1, efb0, efb1, msg0, msg1, agg_sh,
             isem0, isem1, gsem0, gsem1, esem0, esem1, ssem0, ssem1):
    c = lax.axis_index("c")
    s = lax.axis_index("s")

    bufI = (bufI0, bufI1)
    hrow = (hrow0, hrow1)
    efb = (efb0, efb1)
    msg = (msg0, msg1)
    isem = (isem0, isem1)
    gsem = (gsem0, gsem1)
    esem = (esem0, esem1)
    ssem = (ssem0, ssem1)

    pltpu.sync_copy(z_hbm, agg_sh.at[pl.ds(s * STRIPE, STRIPE)])
    plsc.subcore_barrier()

    sd_base = 2 * s * CHUNKS
    ef_base = s * (E_TILE // 4)

    def edge_phase(h_hbm, ef_hbm):
        def idx_row(m, which):
            return bufI[(m // _GRP) % 2].at[2 * (m % _GRP) + which]

        def issue_data(k, m):
            b = m % 2
            g = 16 * k + m
            gd = pltpu.async_copy(h_hbm.at[idx_row(m, 0)], hrow[b], gsem[b])
            ed = pltpu.async_copy(
                ef_hbm.at[pl.ds(ef_base + g * (CB // 4), CB // 4)],
                efb[b], esem[b])
            return gd, ed

        def wait_data(b):
            pltpu.make_async_copy(h_hbm.at[bufI[0].at[0]], hrow[b],
                                  gsem[b]).wait()
            pltpu.make_async_copy(ef_hbm.at[pl.ds(0, CB // 4)], efb[b],
                                  esem[b]).wait()

        def wait_scatter(b):
            pltpu.make_async_copy(msg[b], agg_sh.at[bufI[0].at[1]],
                                  ssem[b]).wait()

        def compute(b):
            @plsc.parallel_loop(0, CB // 4, 1, unroll=2)
            def _(rr):
                for q in range(4):
                    i = 4 * rr + q
                    lo = hrow[b][i, pl.ds(0, 16)] + efb[b][rr, pl.ds(32 * q, 16)]
                    hi = (hrow[b][i, pl.ds(16, 16)]
                          + efb[b][rr, pl.ds(32 * q + 16, 16)])
                    msg[b][i, pl.ds(0, 16)] = jnp.maximum(lo, 0.0)
                    msg[b][i, pl.ds(16, 16)] = jnp.maximum(hi, 0.0)

        pltpu.sync_copy(sd_hbm.at[pl.ds(sd_base, 2 * _GRP)], bufI[0])
        issue_data(0, 0)

        def outer(k, carry):
            for m in range(_BODY):
                b = m % 2
                wait_data(b)
                if m >= 2:
                    wait_scatter(b)
                else:
                    @pl.when(k > 0)
                    def _():
                        wait_scatter(b)
                if m == 2:
                    pltpu.async_copy(
                        sd_hbm.at[pl.ds(sd_base + (32 * k + 16), 2 * _GRP)],
                        bufI[1], isem[1])
                if m == 10:
                    @pl.when(k < _KOUT - 1)
                    def _():
                        pltpu.async_copy(
                            sd_hbm.at[pl.ds(sd_base + (32 * k + 32), 2 * _GRP)],
                            bufI[0], isem[0])
                if m == _GRP - 1:
                    pltpu.make_async_copy(sd_hbm.at[pl.ds(0, 2 * _GRP)],
                                          bufI[1], isem[1]).wait()
                if m == _BODY - 1:
                    @pl.when(k < _KOUT - 1)
                    def _():
                        pltpu.make_async_copy(sd_hbm.at[pl.ds(0, 2 * _GRP)],
                                              bufI[0], isem[0]).wait()
                        issue_data(k + 1, 0)
                else:
                    issue_data(k, m + 1)
                compute(b)
                pltpu.async_copy(msg[b], agg_sh.at[idx_row(m, 1)], ssem[b],
                                 add=True)
            return carry

        lax.fori_loop(0, _KOUT, outer, 0)
        wait_scatter(0)
        wait_scatter(1)

    @pl.when(c == 0)
    def _():
        edge_phase(h0_hbm, ef0_hbm)

    @pl.when(c == 1)
    def _():
        edge_phase(h1_hbm, ef1_hbm)

    plsc.subcore_barrier()

    @pl.when(c == 0)
    def _():
        pltpu.sync_copy(agg_sh.at[pl.ds(s * STRIPE, STRIPE)],
                        o0_hbm.at[pl.ds(s * STRIPE, STRIPE)])

    @pl.when(c == 1)
    def _():
        pltpu.sync_copy(agg_sh.at[pl.ds(s * STRIPE, STRIPE)],
                        o1_hbm.at[pl.ds(s * STRIPE, STRIPE)])


@functools.lru_cache(maxsize=1)
def _sc_aggregate_fn():
    mesh = plsc.VectorSubcoreMesh(
        core_axis_name="c", subcore_axis_name="s",
        num_cores=NC, num_subcores=NS)
    return pl.kernel(
        _sc_body,
        out_type=[jax.ShapeDtypeStruct((N_PAD, H), _f32)] * 2,
        mesh=mesh,
        compiler_params=pltpu.CompilerParams(use_tc_tiling_on_sc=False),
        scratch_types=[
            pltpu.VMEM((2 * _GRP, 128), jnp.int32),
            pltpu.VMEM((2 * _GRP, 128), jnp.int32),
            pltpu.VMEM((CB, H), _f32),
            pltpu.VMEM((CB, H), _f32),
            pltpu.VMEM((CB // 4, 128), _f32),
            pltpu.VMEM((CB // 4, 128), _f32),
            pltpu.VMEM((CB, H), _f32),
            pltpu.VMEM((CB, H), _f32),
            pltpu.VMEM_SHARED((N_PAD, H), _f32),
        ] + [pltpu.SemaphoreType.DMA] * 8,
    )


def _sc_aggregate(*args):
    return _sc_aggregate_fn()(*args)


def kernel(x, edge_index, edge_attr, time_emb, batch,
           node_w1, node_b1, node_w2, node_b2,
           edge_w1, edge_b1, edge_w2, edge_b2,
           time_w, time_b,
           c1_w1, c1_b1, c1_w2, c1_b2,
           c2_w1, c2_b1, c2_w2, c2_b2,
           c3_w1, c3_b1, c3_w2, c3_b2,
           f_w1, f_b1, f_w2, f_b2):
    pad = E_PAD - E
    src = edge_index[0].astype(jnp.int32)
    dst = edge_index[1].astype(jnp.int32)
    src2d = jnp.concatenate([src, jnp.zeros((pad,), jnp.int32)]
                            ).reshape(E_PAD // 128, 128)
    dst2d = jnp.concatenate([dst, jnp.full((pad,), N, jnp.int32)]
                            ).reshape(E_PAD // 128, 128)
    sd = jnp.stack([src2d, dst2d], axis=1).reshape(2 * E_PAD // 128, 128)
    batch2d = batch.astype(jnp.int32).reshape(N, 1)
    zeros_stripe = jnp.zeros((STRIPE, H), _f32)

    ea4 = edge_attr.reshape(E // 4, 16)
    w1p = jnp.zeros((16, 256), _f32)
    w2p = jnp.zeros((256, 256), _f32)
    for q in range(4):
        w1p = w1p.at[4 * q:4 * q + 4, 64 * q:64 * q + 64].set(edge_w1)
        w2p = w2p.at[64 * q:64 * q + 64, 64 * q:64 * q + 64].set(edge_w2)
    b1p = jnp.tile(edge_b1, 4).reshape(1, 256)
    b2p = jnp.tile(edge_b2, 4).reshape(1, 256)

    r2 = lambda b: b.reshape(1, -1)

    h0, h1 = _encoder(x, batch2d, time_emb, time_w, r2(time_b),
                      node_w1, r2(node_b1), node_w2, r2(node_b2))
    ef0, ef1 = _edge_encoder(ea4, w1p, b1p, w2p, b2p)

    for (w1, b1, w2, b2) in ((c1_w1, c1_b1, c1_w2, c1_b2),
                             (c2_w1, c2_b1, c2_w2, c2_b2),
                             (c3_w1, c3_b1, c3_w2, c3_b2)):
        a0, a1 = _sc_aggregate(h0, h1, ef0, ef1, sd, zeros_stripe)
        h0, h1 = _gine_update(h0, h1, a0, a1, w1, r2(b1), w2, r2(b2))

    return _final(h0, h1, f_w1, r2(f_b1), f_w2, f_b2.reshape(1, 1))
